# Initial kernel scaffold; baseline (speedup 1.0000x reference)
import jax
import jax.numpy as jnp
from jax.experimental import pallas as pl


def kernel(x, edge_index, batch, Wlf0, Wrf0, bf0, Wlb0, Wrb0, bb0, Wlf1, Wrf1, bf1, Wlb1, Wrb1, bb1, Wp, bp):
    raise NotImplementedError("write your pallas kernel here")


# speedup vs baseline: 3.9227x; 3.9227x over previous
import functools

import jax
import jax.numpy as jnp
from jax import lax
from jax.experimental import pallas as pl
from jax.experimental.pallas import tpu as pltpu
from jax.experimental.pallas import tpu_sc as plsc

N = 10000
E = 320000
D = 128
G = 64
D_OUT = 64

NS = 16
CHUNK = 80
EPS = E // NS
NCHUNK = EPS // CHUNK
BROW = 80
NBLK = N // BROW
CW = 16


def _sc_aggregate(h, zrows, gidx, sidx, with_counts):
  mesh = plsc.VectorSubcoreMesh(core_axis_name="c", subcore_axis_name="s")
  if with_counts:
    out_type = [jax.ShapeDtypeStruct((2 * N, D), jnp.float32),
                jax.ShapeDtypeStruct((2 * N, D), jnp.float32)]
  else:
    out_type = [jax.ShapeDtypeStruct((2 * N, D), jnp.float32)]
  scratch = [
      pltpu.VMEM((CHUNK,), jnp.int32),
      pltpu.VMEM((CHUNK,), jnp.int32),
      pltpu.VMEM((CHUNK, D), jnp.float32),
      pltpu.VMEM((BROW, D), jnp.float32),
      pltpu.VMEM_SHARED((N, D), jnp.float32),
      pltpu.SemaphoreType.DMA,
  ]

  @functools.partial(
      pl.kernel, mesh=mesh, out_type=out_type, scratch_types=scratch)
  def agg_kernel(h_hbm, z_hbm, gidx_hbm, sidx_hbm, agg_hbm, *rest):
    if with_counts:
      (cnt_hbm, idx_g, idx_s, rows, stage, acc, sem) = rest
    else:
      (idx_g, idx_s, rows, stage, acc, sem) = rest
    c = lax.axis_index("c")
    s = lax.axis_index("s")

    def _per_block(fn):
      @pl.loop(0, pl.cdiv(NBLK, NS))
      def _blk(k):
        j = s + k * NS

        @pl.when(j < NBLK)
        def _():
          fn(j * BROW)

    def _zero_acc():
      pltpu.sync_copy(z_hbm, stage)
      _per_block(lambda lo: pltpu.sync_copy(stage, acc.at[pl.ds(lo, BROW)]))

    def _flush_to(dst_hbm):
      def _one(lo):
        pltpu.sync_copy(acc.at[pl.ds(lo, BROW)], stage)
        pltpu.sync_copy(stage, dst_hbm.at[pl.ds(c * N + lo, BROW)])

      _per_block(_one)

    _zero_acc()
    plsc.subcore_barrier()

    @pl.loop(0, NCHUNK)
    def _chunk(i):
      base = c * E + s * EPS + i * CHUNK
      pltpu.sync_copy(gidx_hbm.at[pl.ds(base, CHUNK)], idx_g)
      pltpu.sync_copy(sidx_hbm.at[pl.ds(base, CHUNK)], idx_s)
      pltpu.async_copy(h_hbm.at[idx_g], rows, sem).wait()
      pltpu.sync_copy(rows, acc.at[idx_s], add=True)

    plsc.subcore_barrier()
    _flush_to(agg_hbm)

    if with_counts:
      plsc.subcore_barrier()
      _zero_acc()

      @pl.loop(0, CHUNK)
      def _fill_ones(r):
        @pl.loop(0, D // 16)
        def _fill16(q):
          rows[r, pl.ds(q * 16, 16)] = jnp.full((16,), 1.0, jnp.float32)

      plsc.subcore_barrier()

      @pl.loop(0, NCHUNK)
      def _cchunk(i):
        base = c * E + s * EPS + i * CHUNK
        pltpu.sync_copy(sidx_hbm.at[pl.ds(base, CHUNK)], idx_s)
        pltpu.sync_copy(rows, acc.at[idx_s], add=True)

      plsc.subcore_barrier()
      _flush_to(cnt_hbm)

  return agg_kernel(h, zrows, gidx, sidx)


ROWB = 1000


def _layer_body(aggf, aggb, cf, cb, h, wlf, wlb, wr, b, out):
  meanf = aggf[...] / jnp.maximum(cf[:, 0:1], 1.0)
  meanb = aggb[...] / jnp.maximum(cb[:, 0:1], 1.0)
  acc = jnp.dot(meanf, wlf[...], preferred_element_type=jnp.float32,
                precision=lax.Precision.HIGHEST)
  acc += jnp.dot(meanb, wlb[...], preferred_element_type=jnp.float32,
                 precision=lax.Precision.HIGHEST)
  acc += jnp.dot(h[...], wr[...], preferred_element_type=jnp.float32,
                 precision=lax.Precision.HIGHEST)
  out[...] = jnp.maximum(acc + b[...], 0.0)


def _layer(aggf, aggb, cf, cb, h, wlf, wlb, wr, b):
  grid = (N // ROWB,)
  row_spec = pl.BlockSpec((ROWB, D), lambda i: (i, 0))
  cnt_spec = pl.BlockSpec((ROWB, CW), lambda i: (i, 0))
  w_spec = pl.BlockSpec((D, D), lambda i: (0, 0))
  b_spec = pl.BlockSpec((1, D), lambda i: (0, 0))
  return pl.pallas_call(
      _layer_body,
      grid=grid,
      in_specs=[row_spec, row_spec, cnt_spec, cnt_spec, row_spec,
                w_spec, w_spec, w_spec, b_spec],
      out_specs=row_spec,
      out_shape=jax.ShapeDtypeStruct((N, D), jnp.float32),
  )(aggf, aggb, cf, cb, h, wlf, wlb, wr, b)


def _final_body(aggf, aggb, cf, cb, h, batch, wlf, wlb, wr, b, wp, bp,
                out, sums, cnts):
  i = pl.program_id(0)

  @pl.when(i == 0)
  def _():
    sums[...] = jnp.zeros_like(sums)
    cnts[...] = jnp.zeros_like(cnts)

  meanf = aggf[...] / jnp.maximum(cf[:, 0:1], 1.0)
  meanb = aggb[...] / jnp.maximum(cb[:, 0:1], 1.0)
  acc = jnp.dot(meanf, wlf[...], preferred_element_type=jnp.float32,
                precision=lax.Precision.HIGHEST)
  acc += jnp.dot(meanb, wlb[...], preferred_element_type=jnp.float32,
                 precision=lax.Precision.HIGHEST)
  acc += jnp.dot(h[...], wr[...], preferred_element_type=jnp.float32,
                 precision=lax.Precision.HIGHEST)
  hblk = jnp.maximum(acc + b[...], 0.0)

  ids = batch[...]
  gids = lax.broadcasted_iota(jnp.int32, (ROWB, G), 1)
  oh = (ids == gids).astype(jnp.float32)
  dn = (((0,), (0,)), ((), ()))
  sums[...] += lax.dot_general(oh, hblk, dn,
                               preferred_element_type=jnp.float32,
                               precision=lax.Precision.HIGHEST)
  cnts[...] += lax.dot_general(oh, jnp.ones((ROWB, D), jnp.float32), dn,
                               preferred_element_type=jnp.float32,
                               precision=lax.Precision.HIGHEST)

  @pl.when(i == pl.num_programs(0) - 1)
  def _():
    pooled = sums[...] / jnp.maximum(cnts[...], 1.0)
    out[...] = jnp.dot(pooled, wp[...], preferred_element_type=jnp.float32,
                       precision=lax.Precision.HIGHEST) + bp[...]


def _final(aggf, aggb, cf, cb, h, batch, wlf, wlb, wr, b, wp, bp):
  grid = (N // ROWB,)
  row_spec = pl.BlockSpec((ROWB, D), lambda i: (i, 0))
  cnt_spec = pl.BlockSpec((ROWB, CW), lambda i: (i, 0))
  batch_spec = pl.BlockSpec((ROWB, 1), lambda i: (i, 0))
  w_spec = pl.BlockSpec((D, D), lambda i: (0, 0))
  b_spec = pl.BlockSpec((1, D), lambda i: (0, 0))
  wp_spec = pl.BlockSpec((D, D_OUT), lambda i: (0, 0))
  bp_spec = pl.BlockSpec((1, D_OUT), lambda i: (0, 0))
  out_spec = pl.BlockSpec((G, D_OUT), lambda i: (0, 0))
  return pl.pallas_call(
      _final_body,
      grid=grid,
      in_specs=[row_spec, row_spec, cnt_spec, cnt_spec, row_spec, batch_spec,
                w_spec, w_spec, w_spec, b_spec, wp_spec, bp_spec],
      out_specs=out_spec,
      out_shape=jax.ShapeDtypeStruct((G, D_OUT), jnp.float32),
      scratch_shapes=[pltpu.VMEM((G, D), jnp.float32),
                      pltpu.VMEM((G, D), jnp.float32)],
  )(aggf, aggb, cf, cb, h, batch, wlf, wlb, wr, b, wp, bp)


def kernel(x, edge_index, batch, Wlf0, Wrf0, bf0, Wlb0, Wrb0, bb0,
           Wlf1, Wrf1, bf1, Wlb1, Wrb1, bb1, Wp, bp):
  src = edge_index[0]
  dst = edge_index[1]
  gidx = jnp.concatenate([src, dst])
  sidx = jnp.concatenate([dst, src])
  zrows = jnp.zeros((BROW, D), jnp.float32)

  agg0, cnt = _sc_aggregate(x, zrows, gidx, sidx, with_counts=True)
  cf, cb = cnt[:N, :CW], cnt[N:, :CW]
  h1 = _layer(agg0[:N], agg0[N:], cf, cb, x,
              Wlf0 * 0.5, Wlb0 * 0.5, (Wrf0 + Wrb0) * 0.5,
              ((bf0 + bb0) * 0.5)[None, :])
  (agg1,) = _sc_aggregate(h1, zrows, gidx, sidx, with_counts=False)
  return _final(agg1[:N], agg1[N:], cf, cb, h1, batch[:, None],
                Wlf1 * 0.5, Wlb1 * 0.5, (Wrf1 + Wrb1) * 0.5,
                ((bf1 + bb1) * 0.5)[None, :], Wp, bp[None, :])


# speedup vs baseline: 6.4718x; 1.6498x over previous
import functools

import jax
import jax.numpy as jnp
from jax import lax
from jax.experimental import pallas as pl
from jax.experimental.pallas import tpu as pltpu
from jax.experimental.pallas import tpu_sc as plsc

N = 10000
E = 320000
D = 128
G = 64
D_OUT = 64

NS = 16
CHUNK = 80
EPS = E // NS
NCHUNK = EPS // CHUNK
BROW = 80
NBLK = N // BROW
NBUF = 4
NGROUP = -(-NCHUNK // NBUF)
CW = 16


def _sc_aggregate(h, zrows, gidx, sidx, with_counts):
  mesh = plsc.VectorSubcoreMesh(core_axis_name="c", subcore_axis_name="s")
  if with_counts:
    out_type = [jax.ShapeDtypeStruct((2 * N, D), jnp.float32),
                jax.ShapeDtypeStruct((2 * N, D), jnp.float32)]
  else:
    out_type = [jax.ShapeDtypeStruct((2 * N, D), jnp.float32)]
  scratch = (
      [pltpu.VMEM((CHUNK,), jnp.int32) for _ in range(NBUF)]
      + [pltpu.VMEM((CHUNK,), jnp.int32) for _ in range(NBUF)]
      + [pltpu.VMEM((CHUNK, D), jnp.float32) for _ in range(NBUF)]
      + [pltpu.VMEM_SHARED((N, D), jnp.float32)]
      + [pltpu.SemaphoreType.DMA for _ in range(NBUF)]
  )

  @functools.partial(
      pl.kernel, mesh=mesh, out_type=out_type, scratch_types=scratch)
  def agg_kernel(h_hbm, z_hbm, gidx_hbm, sidx_hbm, agg_hbm, *rest):
    if with_counts:
      cnt_hbm = rest[0]
      rest = rest[1:]
    idx_g = rest[0:NBUF]
    idx_s = rest[NBUF:2 * NBUF]
    rows = rest[2 * NBUF:3 * NBUF]
    acc = rest[3 * NBUF]
    sems = rest[3 * NBUF + 1:]
    stage = rows[1]
    ones = rows[0]
    c = lax.axis_index("c")
    s = lax.axis_index("s")

    def _per_block(fn):
      @pl.loop(0, pl.cdiv(NBLK, NS))
      def _blk(k):
        j = s + k * NS

        @pl.when(j < NBLK)
        def _():
          fn(j * BROW)

    def _zero_acc():
      pltpu.sync_copy(z_hbm, stage)
      _per_block(lambda lo: pltpu.sync_copy(stage, acc.at[pl.ds(lo, BROW)]))

    def _flush_to(dst_hbm):
      def _one(lo):
        pltpu.sync_copy(acc.at[pl.ds(lo, BROW)], stage)
        pltpu.sync_copy(stage, dst_hbm.at[pl.ds(c * N + lo, BROW)])

      _per_block(_one)

    _zero_acc()
    plsc.subcore_barrier()

    ebase = c * E + s * EPS

    def _fire(i, b):
      pltpu.sync_copy(gidx_hbm.at[pl.ds(ebase + i * CHUNK, CHUNK)], idx_g[b])
      pltpu.sync_copy(sidx_hbm.at[pl.ds(ebase + i * CHUNK, CHUNK)], idx_s[b])
      pltpu.async_copy(h_hbm.at[idx_g[b]], rows[b], sems[b])

    def _drain(b):
      pltpu.make_async_copy(h_hbm.at[idx_g[b]], rows[b], sems[b]).wait()
      pltpu.sync_copy(rows[b], acc.at[idx_s[b]], add=True)

    for b in range(NBUF):
      _fire(b, b)

    @pl.loop(0, NGROUP)
    def _group(g):
      for b in range(NBUF):
        i = g * NBUF + b

        @pl.when(i < NCHUNK)
        def _():
          _drain(b)

        @pl.when(i + NBUF < NCHUNK)
        def _():
          _fire(i + NBUF, b)

    plsc.subcore_barrier()
    _flush_to(agg_hbm)

    if with_counts:
      plsc.subcore_barrier()
      _zero_acc()

      @pl.loop(0, CHUNK)
      def _fill_ones(r):
        @pl.loop(0, D // 16)
        def _fill16(q):
          ones[r, pl.ds(q * 16, 16)] = jnp.full((16,), 1.0, jnp.float32)

      plsc.subcore_barrier()

      def _cfire(i, b):
        pltpu.async_copy(
            sidx_hbm.at[pl.ds(ebase + i * CHUNK, CHUNK)], idx_s[b], sems[b])

      def _cdrain(b):
        pltpu.make_async_copy(
            sidx_hbm.at[pl.ds(0, CHUNK)], idx_s[b], sems[b]).wait()
        pltpu.sync_copy(ones, acc.at[idx_s[b]], add=True)

      for b in range(NBUF):
        _cfire(b, b)

      @pl.loop(0, NGROUP)
      def _cgroup(g):
        for b in range(NBUF):
          i = g * NBUF + b

          @pl.when(i < NCHUNK)
          def _():
            _cdrain(b)

          @pl.when(i + NBUF < NCHUNK)
          def _():
            _cfire(i + NBUF, b)

      plsc.subcore_barrier()
      _flush_to(cnt_hbm)

  return agg_kernel(h, zrows, gidx, sidx)


ROWB = 1000


def _layer_body(aggf, aggb, cf, cb, h, wlf, wlb, wr, b, out):
  meanf = aggf[...] / jnp.maximum(cf[:, 0:1], 1.0)
  meanb = aggb[...] / jnp.maximum(cb[:, 0:1], 1.0)
  acc = jnp.dot(meanf, wlf[...], preferred_element_type=jnp.float32,
                precision=lax.Precision.HIGHEST)
  acc += jnp.dot(meanb, wlb[...], preferred_element_type=jnp.float32,
                 precision=lax.Precision.HIGHEST)
  acc += jnp.dot(h[...], wr[...], preferred_element_type=jnp.float32,
                 precision=lax.Precision.HIGHEST)
  out[...] = jnp.maximum(acc + b[...], 0.0)


def _layer(aggf, aggb, cf, cb, h, wlf, wlb, wr, b):
  grid = (N // ROWB,)
  row_spec = pl.BlockSpec((ROWB, D), lambda i: (i, 0))
  cnt_spec = pl.BlockSpec((ROWB, CW), lambda i: (i, 0))
  w_spec = pl.BlockSpec((D, D), lambda i: (0, 0))
  b_spec = pl.BlockSpec((1, D), lambda i: (0, 0))
  return pl.pallas_call(
      _layer_body,
      grid=grid,
      in_specs=[row_spec, row_spec, cnt_spec, cnt_spec, row_spec,
                w_spec, w_spec, w_spec, b_spec],
      out_specs=row_spec,
      out_shape=jax.ShapeDtypeStruct((N, D), jnp.float32),
  )(aggf, aggb, cf, cb, h, wlf, wlb, wr, b)


def _final_body(aggf, aggb, cf, cb, h, batch, wlf, wlb, wr, b, wp, bp,
                out, sums, cnts):
  i = pl.program_id(0)

  @pl.when(i == 0)
  def _():
    sums[...] = jnp.zeros_like(sums)
    cnts[...] = jnp.zeros_like(cnts)

  meanf = aggf[...] / jnp.maximum(cf[:, 0:1], 1.0)
  meanb = aggb[...] / jnp.maximum(cb[:, 0:1], 1.0)
  acc = jnp.dot(meanf, wlf[...], preferred_element_type=jnp.float32,
                precision=lax.Precision.HIGHEST)
  acc += jnp.dot(meanb, wlb[...], preferred_element_type=jnp.float32,
                 precision=lax.Precision.HIGHEST)
  acc += jnp.dot(h[...], wr[...], preferred_element_type=jnp.float32,
                 precision=lax.Precision.HIGHEST)
  hblk = jnp.maximum(acc + b[...], 0.0)

  ids = batch[...]
  gids = lax.broadcasted_iota(jnp.int32, (ROWB, G), 1)
  oh = (ids == gids).astype(jnp.float32)
  dn = (((0,), (0,)), ((), ()))
  sums[...] += lax.dot_general(oh, hblk, dn,
                               preferred_element_type=jnp.float32,
                               precision=lax.Precision.HIGHEST)
  cnts[...] += lax.dot_general(oh, jnp.ones((ROWB, D), jnp.float32), dn,
                               preferred_element_type=jnp.float32,
                               precision=lax.Precision.HIGHEST)

  @pl.when(i == pl.num_programs(0) - 1)
  def _():
    pooled = sums[...] / jnp.maximum(cnts[...], 1.0)
    out[...] = jnp.dot(pooled, wp[...], preferred_element_type=jnp.float32,
                       precision=lax.Precision.HIGHEST) + bp[...]


def _final(aggf, aggb, cf, cb, h, batch, wlf, wlb, wr, b, wp, bp):
  grid = (N // ROWB,)
  row_spec = pl.BlockSpec((ROWB, D), lambda i: (i, 0))
  cnt_spec = pl.BlockSpec((ROWB, CW), lambda i: (i, 0))
  batch_spec = pl.BlockSpec((ROWB, 1), lambda i: (i, 0))
  w_spec = pl.BlockSpec((D, D), lambda i: (0, 0))
  b_spec = pl.BlockSpec((1, D), lambda i: (0, 0))
  wp_spec = pl.BlockSpec((D, D_OUT), lambda i: (0, 0))
  bp_spec = pl.BlockSpec((1, D_OUT), lambda i: (0, 0))
  out_spec = pl.BlockSpec((G, D_OUT), lambda i: (0, 0))
  return pl.pallas_call(
      _final_body,
      grid=grid,
      in_specs=[row_spec, row_spec, cnt_spec, cnt_spec, row_spec, batch_spec,
                w_spec, w_spec, w_spec, b_spec, wp_spec, bp_spec],
      out_specs=out_spec,
      out_shape=jax.ShapeDtypeStruct((G, D_OUT), jnp.float32),
      scratch_shapes=[pltpu.VMEM((G, D), jnp.float32),
                      pltpu.VMEM((G, D), jnp.float32)],
  )(aggf, aggb, cf, cb, h, batch, wlf, wlb, wr, b, wp, bp)


def kernel(x, edge_index, batch, Wlf0, Wrf0, bf0, Wlb0, Wrb0, bb0,
           Wlf1, Wrf1, bf1, Wlb1, Wrb1, bb1, Wp, bp):
  src = edge_index[0]
  dst = edge_index[1]
  gidx = jnp.concatenate([src, dst])
  sidx = jnp.concatenate([dst, src])
  zrows = jnp.zeros((BROW, D), jnp.float32)

  agg0, cnt = _sc_aggregate(x, zrows, gidx, sidx, with_counts=True)
  cf, cb = cnt[:N, :CW], cnt[N:, :CW]
  h1 = _layer(agg0[:N], agg0[N:], cf, cb, x,
              Wlf0 * 0.5, Wlb0 * 0.5, (Wrf0 + Wrb0) * 0.5,
              ((bf0 + bb0) * 0.5)[None, :])
  (agg1,) = _sc_aggregate(h1, zrows, gidx, sidx, with_counts=False)
  return _final(agg1[:N], agg1[N:], cf, cb, h1, batch[:, None],
                Wlf1 * 0.5, Wlb1 * 0.5, (Wrf1 + Wrb1) * 0.5,
                ((bf1 + bb1) * 0.5)[None, :], Wp, bp[None, :])


# speedup vs baseline: 10.5231x; 1.6260x over previous
import functools

import jax
import jax.numpy as jnp
from jax import lax
from jax.experimental import pallas as pl
from jax.experimental.pallas import tpu as pltpu
from jax.experimental.pallas import tpu_sc as plsc

N = 10000
E = 320000
D = 128
G = 64
D_OUT = 64

NS = 16
CHUNK = 80
EPS = E // NS
NCHUNK = EPS // CHUNK
BROW = 80
NBLK = N // BROW
NBUF = 4
NSLOT = 2 * NBUF
CW = 16


def _sc_aggregate(h, zrows, gidx, sidx, with_counts):
  mesh = plsc.VectorSubcoreMesh(core_axis_name="c", subcore_axis_name="s")
  if with_counts:
    out_type = [jax.ShapeDtypeStruct((2 * N, D), jnp.float32),
                jax.ShapeDtypeStruct((2 * N, D), jnp.float32)]
  else:
    out_type = [jax.ShapeDtypeStruct((2 * N, D), jnp.float32)]
  scratch = (
      [pltpu.VMEM((CHUNK,), jnp.int32) for _ in range(NSLOT)]
      + [pltpu.VMEM((CHUNK,), jnp.int32) for _ in range(NSLOT)]
      + [pltpu.VMEM((CHUNK, D), jnp.float32) for _ in range(NBUF)]
      + [pltpu.VMEM_SHARED((N, D), jnp.float32)]
      + [pltpu.SemaphoreType.DMA for _ in range(NBUF + 2 * NSLOT)]
  )

  @functools.partial(
      pl.kernel, mesh=mesh, out_type=out_type, scratch_types=scratch)
  def agg_kernel(h_hbm, z_hbm, gidx_hbm, sidx_hbm, agg_hbm, *rest):
    if with_counts:
      cnt_hbm = rest[0]
      rest = rest[1:]
    idx_g = rest[0:NSLOT]
    idx_s = rest[NSLOT:2 * NSLOT]
    rows = rest[2 * NSLOT:2 * NSLOT + NBUF]
    acc = rest[2 * NSLOT + NBUF]
    sems = rest[2 * NSLOT + NBUF + 1:]
    gsem = sems[0:NBUF]
    igsem = sems[NBUF:NBUF + NSLOT]
    issem = sems[NBUF + NSLOT:]
    c = lax.axis_index("c")
    s = lax.axis_index("s")
    stage = rows[1]
    ones = rows[0]

    def _per_block(fn):
      @pl.loop(0, pl.cdiv(NBLK, NS))
      def _blk(k):
        j = s + k * NS

        @pl.when(j < NBLK)
        def _():
          fn(j * BROW)

    def _zero_acc():
      pltpu.sync_copy(z_hbm, stage)
      _per_block(lambda lo: pltpu.sync_copy(stage, acc.at[pl.ds(lo, BROW)]))

    def _flush_to(dst_hbm):
      def _one(lo):
        pltpu.sync_copy(acc.at[pl.ds(lo, BROW)], stage)
        pltpu.sync_copy(stage, dst_hbm.at[pl.ds(c * N + lo, BROW)])

      _per_block(_one)

    _zero_acc()
    plsc.subcore_barrier()

    ebase = c * E + s * EPS

    def _idx_fire(i, sl):
      pltpu.async_copy(
          gidx_hbm.at[pl.ds(ebase + i * CHUNK, CHUNK)], idx_g[sl], igsem[sl])
      pltpu.async_copy(
          sidx_hbm.at[pl.ds(ebase + i * CHUNK, CHUNK)], idx_s[sl], issem[sl])

    def _gather_fire(b, sl):
      pltpu.make_async_copy(
          gidx_hbm.at[pl.ds(0, CHUNK)], idx_g[sl], igsem[sl]).wait()
      pltpu.make_async_copy(
          sidx_hbm.at[pl.ds(0, CHUNK)], idx_s[sl], issem[sl]).wait()
      pltpu.async_copy(h_hbm.at[idx_g[sl]], rows[b], gsem[b])

    def _drain(b, sl):
      pltpu.make_async_copy(h_hbm.at[pl.ds(0, CHUNK)], rows[b], gsem[b]).wait()
      pltpu.sync_copy(rows[b], acc.at[idx_s[sl]], add=True)

    for i in range(NSLOT):
      _idx_fire(i, i)
    for i in range(NBUF):
      _gather_fire(i, i)

    @pl.loop(0, pl.cdiv(NCHUNK, NSLOT))
    def _group(g):
      for u in range(NSLOT):
        i = g * NSLOT + u
        b = u % NBUF

        @pl.when(i < NCHUNK)
        def _():
          _drain(b, u)

        @pl.when(i + NBUF < NCHUNK)
        def _():
          _gather_fire(b, (u + NBUF) % NSLOT)

        @pl.when(i + NSLOT < NCHUNK)
        def _():
          _idx_fire(i + NSLOT, u)

    plsc.subcore_barrier()
    _flush_to(agg_hbm)

    if with_counts:
      plsc.subcore_barrier()
      _zero_acc()

      @pl.loop(0, CHUNK)
      def _fill_ones(r):
        @pl.loop(0, D // 16)
        def _fill16(q):
          ones[r, pl.ds(q * 16, 16)] = jnp.full((16,), 1.0, jnp.float32)

      plsc.subcore_barrier()

      def _cfire(i, sl):
        pltpu.async_copy(
            sidx_hbm.at[pl.ds(ebase + i * CHUNK, CHUNK)], idx_s[sl], issem[sl])

      def _cdrain(sl):
        pltpu.make_async_copy(
            sidx_hbm.at[pl.ds(0, CHUNK)], idx_s[sl], issem[sl]).wait()
        pltpu.sync_copy(ones, acc.at[idx_s[sl]], add=True)

      for i in range(NSLOT):
        _cfire(i, i)

      @pl.loop(0, pl.cdiv(NCHUNK, NSLOT))
      def _cgroup(g):
        for u in range(NSLOT):
          i = g * NSLOT + u

          @pl.when(i < NCHUNK)
          def _():
            _cdrain(u)

          @pl.when(i + NSLOT < NCHUNK)
          def _():
            _cfire(i + NSLOT, u)

      plsc.subcore_barrier()
      _flush_to(cnt_hbm)

  return agg_kernel(h, zrows, gidx, sidx)


ROWB = 1000


def _layer_body(aggf, aggb, cf, cb, h, wlf, wlb, wr, b, out):
  meanf = aggf[...] / jnp.maximum(cf[:, 0:1], 1.0)
  meanb = aggb[...] / jnp.maximum(cb[:, 0:1], 1.0)
  acc = jnp.dot(meanf, wlf[...], preferred_element_type=jnp.float32,
                precision=lax.Precision.HIGHEST)
  acc += jnp.dot(meanb, wlb[...], preferred_element_type=jnp.float32,
                 precision=lax.Precision.HIGHEST)
  acc += jnp.dot(h[...], wr[...], preferred_element_type=jnp.float32,
                 precision=lax.Precision.HIGHEST)
  out[...] = jnp.maximum(acc + b[...], 0.0)


def _layer(aggf, aggb, cf, cb, h, wlf, wlb, wr, b):
  grid = (N // ROWB,)
  row_spec = pl.BlockSpec((ROWB, D), lambda i: (i, 0))
  cnt_spec = pl.BlockSpec((ROWB, CW), lambda i: (i, 0))
  w_spec = pl.BlockSpec((D, D), lambda i: (0, 0))
  b_spec = pl.BlockSpec((1, D), lambda i: (0, 0))
  return pl.pallas_call(
      _layer_body,
      grid=grid,
      in_specs=[row_spec, row_spec, cnt_spec, cnt_spec, row_spec,
                w_spec, w_spec, w_spec, b_spec],
      out_specs=row_spec,
      out_shape=jax.ShapeDtypeStruct((N, D), jnp.float32),
  )(aggf, aggb, cf, cb, h, wlf, wlb, wr, b)


def _final_body(aggf, aggb, cf, cb, h, batch, wlf, wlb, wr, b, wp, bp,
                out, sums, cnts):
  i = pl.program_id(0)

  @pl.when(i == 0)
  def _():
    sums[...] = jnp.zeros_like(sums)
    cnts[...] = jnp.zeros_like(cnts)

  meanf = aggf[...] / jnp.maximum(cf[:, 0:1], 1.0)
  meanb = aggb[...] / jnp.maximum(cb[:, 0:1], 1.0)
  acc = jnp.dot(meanf, wlf[...], preferred_element_type=jnp.float32,
                precision=lax.Precision.HIGHEST)
  acc += jnp.dot(meanb, wlb[...], preferred_element_type=jnp.float32,
                 precision=lax.Precision.HIGHEST)
  acc += jnp.dot(h[...], wr[...], preferred_element_type=jnp.float32,
                 precision=lax.Precision.HIGHEST)
  hblk = jnp.maximum(acc + b[...], 0.0)

  ids = batch[...]
  gids = lax.broadcasted_iota(jnp.int32, (ROWB, G), 1)
  oh = (ids == gids).astype(jnp.float32)
  dn = (((0,), (0,)), ((), ()))
  sums[...] += lax.dot_general(oh, hblk, dn,
                               preferred_element_type=jnp.float32,
                               precision=lax.Precision.HIGHEST)
  cnts[...] += lax.dot_general(oh, jnp.ones((ROWB, D), jnp.float32), dn,
                               preferred_element_type=jnp.float32,
                               precision=lax.Precision.HIGHEST)

  @pl.when(i == pl.num_programs(0) - 1)
  def _():
    pooled = sums[...] / jnp.maximum(cnts[...], 1.0)
    out[...] = jnp.dot(pooled, wp[...], preferred_element_type=jnp.float32,
                       precision=lax.Precision.HIGHEST) + bp[...]


def _final(aggf, aggb, cf, cb, h, batch, wlf, wlb, wr, b, wp, bp):
  grid = (N // ROWB,)
  row_spec = pl.BlockSpec((ROWB, D), lambda i: (i, 0))
  cnt_spec = pl.BlockSpec((ROWB, CW), lambda i: (i, 0))
  batch_spec = pl.BlockSpec((ROWB, 1), lambda i: (i, 0))
  w_spec = pl.BlockSpec((D, D), lambda i: (0, 0))
  b_spec = pl.BlockSpec((1, D), lambda i: (0, 0))
  wp_spec = pl.BlockSpec((D, D_OUT), lambda i: (0, 0))
  bp_spec = pl.BlockSpec((1, D_OUT), lambda i: (0, 0))
  out_spec = pl.BlockSpec((G, D_OUT), lambda i: (0, 0))
  return pl.pallas_call(
      _final_body,
      grid=grid,
      in_specs=[row_spec, row_spec, cnt_spec, cnt_spec, row_spec, batch_spec,
                w_spec, w_spec, w_spec, b_spec, wp_spec, bp_spec],
      out_specs=out_spec,
      out_shape=jax.ShapeDtypeStruct((G, D_OUT), jnp.float32),
      scratch_shapes=[pltpu.VMEM((G, D), jnp.float32),
                      pltpu.VMEM((G, D), jnp.float32)],
  )(aggf, aggb, cf, cb, h, batch, wlf, wlb, wr, b, wp, bp)


def kernel(x, edge_index, batch, Wlf0, Wrf0, bf0, Wlb0, Wrb0, bb0,
           Wlf1, Wrf1, bf1, Wlb1, Wrb1, bb1, Wp, bp):
  src = edge_index[0]
  dst = edge_index[1]
  gidx = jnp.concatenate([src, dst])
  sidx = jnp.concatenate([dst, src])
  zrows = jnp.zeros((BROW, D), jnp.float32)

  agg0, cnt = _sc_aggregate(x, zrows, gidx, sidx, with_counts=True)
  cf, cb = cnt[:N, :CW], cnt[N:, :CW]
  h1 = _layer(agg0[:N], agg0[N:], cf, cb, x,
              Wlf0 * 0.5, Wlb0 * 0.5, (Wrf0 + Wrb0) * 0.5,
              ((bf0 + bb0) * 0.5)[None, :])
  (agg1,) = _sc_aggregate(h1, zrows, gidx, sidx, with_counts=False)
  return _final(agg1[:N], agg1[N:], cf, cb, h1, batch[:, None],
                Wlf1 * 0.5, Wlb1 * 0.5, (Wrf1 + Wrb1) * 0.5,
                ((bf1 + bb1) * 0.5)[None, :], Wp, bp[None, :])


# speedup vs baseline: 13.3096x; 1.2648x over previous
import dataclasses
import functools

import jax
import jax.numpy as jnp
from jax import lax
from jax.experimental import pallas as pl
from jax.experimental.pallas import tpu as pltpu
from jax.experimental.pallas import tpu_sc as plsc

N = 10000
E = 320000
D = 128
G = 64
D_OUT = 64

NS = 16
CHUNK = 80
EPS = E // NS
NCHUNK = EPS // CHUNK
BROW = 80
NBLK = N // BROW
NBUF = 4
NSLOT = 2 * NBUF
HR = 80


def _sc_aggregate(h, zrows, eidx, with_counts):
  nbuf = 3 if with_counts else 4
  nslot = 2 * nbuf
  mesh = plsc.VectorSubcoreMesh(core_axis_name="c", subcore_axis_name="s")
  if with_counts:
    out_type = [jax.ShapeDtypeStruct((2 * N, D), jnp.float32),
                jax.ShapeDtypeStruct((2 * HR, D), jnp.float32)]
  else:
    out_type = [jax.ShapeDtypeStruct((2 * N, D), jnp.float32)]
  scratch = (
      [pltpu.VMEM((CHUNK,), jnp.int32) for _ in range(nslot)]
      + [pltpu.VMEM((CHUNK,), jnp.int32) for _ in range(nslot)]
      + [pltpu.VMEM((CHUNK, D), jnp.float32) for _ in range(nbuf)]
      + [pltpu.VMEM_SHARED((N, D), jnp.float32)]
      + [pltpu.SemaphoreType.DMA for _ in range(nbuf + 2 * nslot)]
      + ([pltpu.VMEM((HR, D), jnp.float32),
          pltpu.VMEM((HR,), jnp.int32),
          pltpu.VMEM_SHARED((HR, D), jnp.float32)]
         if with_counts else [])
  )

  cp = pltpu.CompilerParams()
  if with_counts and "needs_layout_passes" in pltpu.CompilerParams.__dataclass_fields__:
    cp = dataclasses.replace(cp, needs_layout_passes=False)

  @functools.partial(
      pl.kernel, mesh=mesh, out_type=out_type, scratch_types=scratch,
      compiler_params=cp)
  def agg_kernel(h_hbm, z_hbm, eidx_hbm, agg_hbm, *rest):
    if with_counts:
      cnt_hbm = rest[0]
      hist, rowids, cacc = rest[-3:]
      rest = rest[1:-3]
    idx_g = rest[0:nslot]
    idx_s = rest[nslot:2 * nslot]
    rows = rest[2 * nslot:2 * nslot + nbuf]
    acc = rest[2 * nslot + nbuf]
    sems = rest[2 * nslot + nbuf + 1:]
    gsem = sems[0:nbuf]
    igsem = sems[nbuf:nbuf + nslot]
    issem = sems[nbuf + nslot:]
    c = lax.axis_index("c")
    s = lax.axis_index("s")
    stage = rows[1]

    def _per_block(fn):
      @pl.loop(0, pl.cdiv(NBLK, NS))
      def _blk(k):
        j = s + k * NS

        @pl.when(j < NBLK)
        def _():
          fn(j * BROW)

    def _zero_acc():
      pltpu.sync_copy(z_hbm, stage)
      _per_block(lambda lo: pltpu.sync_copy(stage, acc.at[pl.ds(lo, BROW)]))

    def _flush_to(dst_hbm):
      def _one(lo):
        pltpu.sync_copy(acc.at[pl.ds(lo, BROW)], stage)
        pltpu.sync_copy(stage, dst_hbm.at[pl.ds(c * N + lo, BROW)])

      _per_block(_one)

    _zero_acc()
    if with_counts:
      @pl.when(s == 0)
      def _():
        pltpu.sync_copy(stage, cacc)

      pltpu.sync_copy(z_hbm, hist)

      @pl.loop(0, HR // 16)
      def _iota16(r):
        rowids[pl.ds(r * 16, 16)] = (
            lax.iota(jnp.int32, 16) + jnp.full((16,), 1, jnp.int32) * r * 16)

    plsc.subcore_barrier()

    gbase = c * E + s * EPS
    sbase = (1 - c) * E + s * EPS

    def _idx_fire(i, sl):
      pltpu.async_copy(
          eidx_hbm.at[pl.ds(gbase + i * CHUNK, CHUNK)], idx_g[sl], igsem[sl])
      pltpu.async_copy(
          eidx_hbm.at[pl.ds(sbase + i * CHUNK, CHUNK)], idx_s[sl], issem[sl])

    def _gather_fire(b, sl):
      pltpu.make_async_copy(
          eidx_hbm.at[pl.ds(0, CHUNK)], idx_g[sl], igsem[sl]).wait()
      pltpu.make_async_copy(
          eidx_hbm.at[pl.ds(0, CHUNK)], idx_s[sl], issem[sl]).wait()
      pltpu.async_copy(h_hbm.at[idx_g[sl]], rows[b], gsem[b])

    def _drain(b, sl):
      pltpu.make_async_copy(h_hbm.at[pl.ds(0, CHUNK)], rows[b], gsem[b]).wait()
      pltpu.sync_copy(rows[b], acc.at[idx_s[sl]], add=True)
      if with_counts:
        one16 = jnp.full((16,), 1.0, jnp.float32)
        for q in range(CHUNK // 16):
          nid = idx_s[sl][pl.ds(q * 16, 16)]
          plsc.addupdate_scatter(
              hist, [lax.shift_right_logical(nid, jnp.full((16,), 7, jnp.int32)),
                     lax.bitwise_and(nid, jnp.full((16,), D - 1, jnp.int32))],
              one16)

    for i in range(nslot):
      _idx_fire(i, i)
    for i in range(nbuf):
      _gather_fire(i, i)

    @pl.loop(0, pl.cdiv(NCHUNK, nslot))
    def _group(g):
      for u in range(nslot):
        i = g * nslot + u
        b = u % nbuf

        @pl.when(i < NCHUNK)
        def _():
          _drain(b, u)

        @pl.when(i + nbuf < NCHUNK)
        def _():
          _gather_fire(b, (u + nbuf) % nslot)

        @pl.when(i + nslot < NCHUNK)
        def _():
          _idx_fire(i + nslot, u)

    plsc.subcore_barrier()
    _flush_to(agg_hbm)

    if with_counts:
      pltpu.sync_copy(hist, cacc.at[rowids], add=True)
      plsc.subcore_barrier()

      @pl.when(s == 0)
      def _():
        pltpu.sync_copy(cacc, stage)
        pltpu.sync_copy(stage, cnt_hbm.at[pl.ds(c * HR, HR)])

  return agg_kernel(h, zrows, eidx)


ROWB = 1000


def _layer_body(aggf, aggb, cf, cb, h, wlf, wlb, wr, b, out):
  meanf = aggf[...] / jnp.maximum(cf[:, 0:1], 1.0)
  meanb = aggb[...] / jnp.maximum(cb[:, 0:1], 1.0)
  acc = jnp.dot(meanf, wlf[...], preferred_element_type=jnp.float32,
                precision=lax.Precision.HIGHEST)
  acc += jnp.dot(meanb, wlb[...], preferred_element_type=jnp.float32,
                 precision=lax.Precision.HIGHEST)
  acc += jnp.dot(h[...], wr[...], preferred_element_type=jnp.float32,
                 precision=lax.Precision.HIGHEST)
  out[...] = jnp.maximum(acc + b[...], 0.0)


def _layer(aggf, aggb, cf, cb, h, wlf, wlb, wr, b):
  grid = (N // ROWB,)
  row_spec = pl.BlockSpec((ROWB, D), lambda i: (i, 0))
  cnt_spec = pl.BlockSpec((ROWB, 1), lambda i: (i, 0))
  w_spec = pl.BlockSpec((D, D), lambda i: (0, 0))
  b_spec = pl.BlockSpec((1, D), lambda i: (0, 0))
  return pl.pallas_call(
      _layer_body,
      grid=grid,
      in_specs=[row_spec, row_spec, cnt_spec, cnt_spec, row_spec,
                w_spec, w_spec, w_spec, b_spec],
      out_specs=row_spec,
      out_shape=jax.ShapeDtypeStruct((N, D), jnp.float32),
  )(aggf, aggb, cf, cb, h, wlf, wlb, wr, b)


def _final_body(aggf, aggb, cf, cb, h, batch, wlf, wlb, wr, b, wp, bp,
                out, sums, cnts):
  i = pl.program_id(0)

  @pl.when(i == 0)
  def _():
    sums[...] = jnp.zeros_like(sums)
    cnts[...] = jnp.zeros_like(cnts)

  meanf = aggf[...] / jnp.maximum(cf[:, 0:1], 1.0)
  meanb = aggb[...] / jnp.maximum(cb[:, 0:1], 1.0)
  acc = jnp.dot(meanf, wlf[...], preferred_element_type=jnp.float32,
                precision=lax.Precision.HIGHEST)
  acc += jnp.dot(meanb, wlb[...], preferred_element_type=jnp.float32,
                 precision=lax.Precision.HIGHEST)
  acc += jnp.dot(h[...], wr[...], preferred_element_type=jnp.float32,
                 precision=lax.Precision.HIGHEST)
  hblk = jnp.maximum(acc + b[...], 0.0)

  ids = batch[...]
  gids = lax.broadcasted_iota(jnp.int32, (ROWB, G), 1)
  oh = (ids == gids).astype(jnp.float32)
  dn = (((0,), (0,)), ((), ()))
  sums[...] += lax.dot_general(oh, hblk, dn,
                               preferred_element_type=jnp.float32,
                               precision=lax.Precision.HIGHEST)
  cnts[...] += lax.dot_general(oh, jnp.ones((ROWB, D), jnp.float32), dn,
                               preferred_element_type=jnp.float32,
                               precision=lax.Precision.HIGHEST)

  @pl.when(i == pl.num_programs(0) - 1)
  def _():
    pooled = sums[...] / jnp.maximum(cnts[...], 1.0)
    out[...] = jnp.dot(pooled, wp[...], preferred_element_type=jnp.float32,
                       precision=lax.Precision.HIGHEST) + bp[...]


def _final(aggf, aggb, cf, cb, h, batch, wlf, wlb, wr, b, wp, bp):
  grid = (N // ROWB,)
  row_spec = pl.BlockSpec((ROWB, D), lambda i: (i, 0))
  cnt_spec = pl.BlockSpec((ROWB, 1), lambda i: (i, 0))
  batch_spec = pl.BlockSpec((ROWB, 1), lambda i: (i, 0))
  w_spec = pl.BlockSpec((D, D), lambda i: (0, 0))
  b_spec = pl.BlockSpec((1, D), lambda i: (0, 0))
  wp_spec = pl.BlockSpec((D, D_OUT), lambda i: (0, 0))
  bp_spec = pl.BlockSpec((1, D_OUT), lambda i: (0, 0))
  out_spec = pl.BlockSpec((G, D_OUT), lambda i: (0, 0))
  return pl.pallas_call(
      _final_body,
      grid=grid,
      in_specs=[row_spec, row_spec, cnt_spec, cnt_spec, row_spec, batch_spec,
                w_spec, w_spec, w_spec, b_spec, wp_spec, bp_spec],
      out_specs=out_spec,
      out_shape=jax.ShapeDtypeStruct((G, D_OUT), jnp.float32),
      scratch_shapes=[pltpu.VMEM((G, D), jnp.float32),
                      pltpu.VMEM((G, D), jnp.float32)],
  )(aggf, aggb, cf, cb, h, batch, wlf, wlb, wr, b, wp, bp)


def kernel(x, edge_index, batch, Wlf0, Wrf0, bf0, Wlb0, Wrb0, bb0,
           Wlf1, Wrf1, bf1, Wlb1, Wrb1, bb1, Wp, bp):
  eidx = edge_index.reshape(2 * E)
  zrows = jnp.zeros((BROW, D), jnp.float32)

  agg0, cnt = _sc_aggregate(x, zrows, eidx, with_counts=True)
  cf = cnt[:HR].reshape(HR * D)[:N, None]
  cb = cnt[HR:].reshape(HR * D)[:N, None]
  h1 = _layer(agg0[:N], agg0[N:], cf, cb, x,
              Wlf0 * 0.5, Wlb0 * 0.5, (Wrf0 + Wrb0) * 0.5,
              ((bf0 + bb0) * 0.5)[None, :])
  (agg1,) = _sc_aggregate(h1, zrows, eidx, with_counts=False)
  return _final(agg1[:N], agg1[N:], cf, cb, h1, batch[:, None],
                Wlf1 * 0.5, Wlb1 * 0.5, (Wrf1 + Wrb1) * 0.5,
                ((bf1 + bb1) * 0.5)[None, :], Wp, bp[None, :])


# speedup vs baseline: 13.6326x; 1.0243x over previous
import dataclasses
import functools

import jax
import jax.numpy as jnp
from jax import lax
from jax.experimental import pallas as pl
from jax.experimental.pallas import tpu as pltpu
from jax.experimental.pallas import tpu_sc as plsc

N = 10000
E = 320000
D = 128
G = 64
D_OUT = 64

NS = 16
CHUNK = 80
EPS = E // NS
NCHUNK = EPS // CHUNK
BROW = 80
NBLK = N // BROW
NBUF = 4
NSLOT = 2 * NBUF
HR = 80


def _sc_aggregate(h, zrows, eidx, with_counts):
  nbuf = 3 if with_counts else 4
  nslot = 2 * nbuf
  mesh = plsc.VectorSubcoreMesh(core_axis_name="c", subcore_axis_name="s")
  if with_counts:
    out_type = [jax.ShapeDtypeStruct((2 * N, D), jnp.float32),
                jax.ShapeDtypeStruct((2 * HR, D), jnp.float32)]
  else:
    out_type = [jax.ShapeDtypeStruct((2 * N, D), jnp.float32)]
  scratch = (
      [pltpu.VMEM((CHUNK,), jnp.int32) for _ in range(nslot)]
      + [pltpu.VMEM((CHUNK,), jnp.int32) for _ in range(nslot)]
      + [pltpu.VMEM((CHUNK, D), jnp.float32) for _ in range(nbuf)]
      + [pltpu.VMEM_SHARED((N, D), jnp.float32)]
      + [pltpu.SemaphoreType.DMA for _ in range(nbuf + 2 * nslot)]
      + ([pltpu.VMEM((HR, D), jnp.float32),
          pltpu.VMEM((HR,), jnp.int32),
          pltpu.VMEM_SHARED((HR, D), jnp.float32)]
         if with_counts else [])
  )

  cp = pltpu.CompilerParams()
  if with_counts and "needs_layout_passes" in pltpu.CompilerParams.__dataclass_fields__:
    cp = dataclasses.replace(cp, needs_layout_passes=False)

  @functools.partial(
      pl.kernel, mesh=mesh, out_type=out_type, scratch_types=scratch,
      compiler_params=cp)
  def agg_kernel(h_hbm, z_hbm, eidx_hbm, agg_hbm, *rest):
    if with_counts:
      cnt_hbm = rest[0]
      hist, rowids, cacc = rest[-3:]
      rest = rest[1:-3]
    idx_g = rest[0:nslot]
    idx_s = rest[nslot:2 * nslot]
    rows = rest[2 * nslot:2 * nslot + nbuf]
    acc = rest[2 * nslot + nbuf]
    sems = rest[2 * nslot + nbuf + 1:]
    gsem = sems[0:nbuf]
    igsem = sems[nbuf:nbuf + nslot]
    issem = sems[nbuf + nslot:]
    c = lax.axis_index("c")
    s = lax.axis_index("s")
    stage = rows[1]

    def _per_block(fn):
      @pl.loop(0, pl.cdiv(NBLK, NS))
      def _blk(k):
        j = s + k * NS

        @pl.when(j < NBLK)
        def _():
          fn(j * BROW)

    def _zero_acc():
      pltpu.sync_copy(z_hbm, stage)
      _per_block(lambda lo: pltpu.sync_copy(stage, acc.at[pl.ds(lo, BROW)]))

    def _flush_to(dst_hbm):
      def _one(lo):
        pltpu.sync_copy(acc.at[pl.ds(lo, BROW)], stage)
        pltpu.sync_copy(stage, dst_hbm.at[pl.ds(c * N + lo, BROW)])

      _per_block(_one)

    _zero_acc()
    if with_counts:
      @pl.when(s == 0)
      def _():
        pltpu.sync_copy(stage, cacc)

      pltpu.sync_copy(z_hbm, hist)

      @pl.loop(0, HR // 16)
      def _iota16(r):
        rowids[pl.ds(r * 16, 16)] = (
            lax.iota(jnp.int32, 16) + jnp.full((16,), 1, jnp.int32) * r * 16)

    plsc.subcore_barrier()

    gbase = c * E + s * EPS
    sbase = (1 - c) * E + s * EPS

    def _idx_fire(i, sl):
      pltpu.async_copy(
          eidx_hbm.at[pl.ds(gbase + i * CHUNK, CHUNK)], idx_g[sl], igsem[sl])
      pltpu.async_copy(
          eidx_hbm.at[pl.ds(sbase + i * CHUNK, CHUNK)], idx_s[sl], issem[sl])

    def _gather_fire(b, sl):
      pltpu.make_async_copy(
          eidx_hbm.at[pl.ds(0, CHUNK)], idx_g[sl], igsem[sl]).wait()
      pltpu.make_async_copy(
          eidx_hbm.at[pl.ds(0, CHUNK)], idx_s[sl], issem[sl]).wait()
      pltpu.async_copy(h_hbm.at[idx_g[sl]], rows[b], gsem[b])

    def _drain(b, sl):
      pltpu.make_async_copy(h_hbm.at[pl.ds(0, CHUNK)], rows[b], gsem[b]).wait()
      pltpu.sync_copy(rows[b], acc.at[idx_s[sl]], add=True)
      if with_counts:
        one16 = jnp.full((16,), 1.0, jnp.float32)
        for q in range(CHUNK // 16):
          nid = idx_s[sl][pl.ds(q * 16, 16)]
          plsc.addupdate_scatter(
              hist, [lax.shift_right_logical(nid, jnp.full((16,), 7, jnp.int32)),
                     lax.bitwise_and(nid, jnp.full((16,), D - 1, jnp.int32))],
              one16)

    for i in range(nslot):
      _idx_fire(i, i)
    for i in range(nbuf):
      _gather_fire(i, i)

    @pl.loop(0, pl.cdiv(NCHUNK, nslot))
    def _group(g):
      for u in range(nslot):
        i = g * nslot + u
        b = u % nbuf

        @pl.when(i < NCHUNK)
        def _():
          _drain(b, u)

        @pl.when(i + nbuf < NCHUNK)
        def _():
          _gather_fire(b, (u + nbuf) % nslot)

        @pl.when(i + nslot < NCHUNK)
        def _():
          _idx_fire(i + nslot, u)

    plsc.subcore_barrier()
    _flush_to(agg_hbm)

    if with_counts:
      pltpu.sync_copy(hist, cacc.at[rowids], add=True)
      plsc.subcore_barrier()

      @pl.when(s == 0)
      def _():
        pltpu.sync_copy(cacc, stage)
        pltpu.sync_copy(stage, cnt_hbm.at[pl.ds(c * HR, HR)])

  return agg_kernel(h, zrows, eidx)


ROWB = 1000


def _layer_body(aggf, aggb, cf, cb, h, wlf, wlb, wr, b, out):
  meanf = aggf[...] / jnp.maximum(cf[:, 0:1], 1.0)
  meanb = aggb[...] / jnp.maximum(cb[:, 0:1], 1.0)
  acc = jnp.dot(meanf, wlf[...], preferred_element_type=jnp.float32,
                precision=lax.Precision.HIGHEST)
  acc += jnp.dot(meanb, wlb[...], preferred_element_type=jnp.float32,
                 precision=lax.Precision.HIGHEST)
  acc += jnp.dot(h[...], wr[...], preferred_element_type=jnp.float32,
                 precision=lax.Precision.HIGHEST)
  out[...] = jnp.maximum(acc + b[...], 0.0)


def _layer(agg, cf, cb, h, wlf, wlb, wr, b):
  grid = (N // ROWB,)
  row_spec = pl.BlockSpec((ROWB, D), lambda i: (i, 0))
  aggb_spec = pl.BlockSpec((ROWB, D), lambda i: (i + N // ROWB, 0))
  cnt_spec = pl.BlockSpec((ROWB, 1), lambda i: (i, 0))
  w_spec = pl.BlockSpec((D, D), lambda i: (0, 0))
  b_spec = pl.BlockSpec((1, D), lambda i: (0, 0))
  return pl.pallas_call(
      _layer_body,
      grid=grid,
      in_specs=[row_spec, aggb_spec, cnt_spec, cnt_spec, row_spec,
                w_spec, w_spec, w_spec, b_spec],
      out_specs=row_spec,
      out_shape=jax.ShapeDtypeStruct((N, D), jnp.float32),
  )(agg, agg, cf, cb, h, wlf, wlb, wr, b)


def _final_body(aggf, aggb, cf, cb, h, batch, wlf, wlb, wr, b, wp, bp,
                out, sums, cnts):
  i = pl.program_id(0)

  @pl.when(i == 0)
  def _():
    sums[...] = jnp.zeros_like(sums)
    cnts[...] = jnp.zeros_like(cnts)

  meanf = aggf[...] / jnp.maximum(cf[:, 0:1], 1.0)
  meanb = aggb[...] / jnp.maximum(cb[:, 0:1], 1.0)
  acc = jnp.dot(meanf, wlf[...], preferred_element_type=jnp.float32,
                precision=lax.Precision.HIGHEST)
  acc += jnp.dot(meanb, wlb[...], preferred_element_type=jnp.float32,
                 precision=lax.Precision.HIGHEST)
  acc += jnp.dot(h[...], wr[...], preferred_element_type=jnp.float32,
                 precision=lax.Precision.HIGHEST)
  hblk = jnp.maximum(acc + b[...], 0.0)

  ids = batch[...]
  gids = lax.broadcasted_iota(jnp.int32, (ROWB, G), 1)
  oh = (ids == gids).astype(jnp.float32)
  dn = (((0,), (0,)), ((), ()))
  sums[...] += lax.dot_general(oh, hblk, dn,
                               preferred_element_type=jnp.float32,
                               precision=lax.Precision.HIGHEST)
  cnts[...] += lax.dot_general(oh, jnp.ones((ROWB, D), jnp.float32), dn,
                               preferred_element_type=jnp.float32,
                               precision=lax.Precision.HIGHEST)

  @pl.when(i == pl.num_programs(0) - 1)
  def _():
    pooled = sums[...] / jnp.maximum(cnts[...], 1.0)
    out[...] = jnp.dot(pooled, wp[...], preferred_element_type=jnp.float32,
                       precision=lax.Precision.HIGHEST) + bp[...]


def _final(agg, cf, cb, h, batch, wlf, wlb, wr, b, wp, bp):
  grid = (N // ROWB,)
  row_spec = pl.BlockSpec((ROWB, D), lambda i: (i, 0))
  aggb_spec = pl.BlockSpec((ROWB, D), lambda i: (i + N // ROWB, 0))
  cnt_spec = pl.BlockSpec((ROWB, 1), lambda i: (i, 0))
  batch_spec = pl.BlockSpec((ROWB, 1), lambda i: (i, 0))
  w_spec = pl.BlockSpec((D, D), lambda i: (0, 0))
  b_spec = pl.BlockSpec((1, D), lambda i: (0, 0))
  wp_spec = pl.BlockSpec((D, D_OUT), lambda i: (0, 0))
  bp_spec = pl.BlockSpec((1, D_OUT), lambda i: (0, 0))
  out_spec = pl.BlockSpec((G, D_OUT), lambda i: (0, 0))
  return pl.pallas_call(
      _final_body,
      grid=grid,
      in_specs=[row_spec, aggb_spec, cnt_spec, cnt_spec, row_spec, batch_spec,
                w_spec, w_spec, w_spec, b_spec, wp_spec, bp_spec],
      out_specs=out_spec,
      out_shape=jax.ShapeDtypeStruct((G, D_OUT), jnp.float32),
      scratch_shapes=[pltpu.VMEM((G, D), jnp.float32),
                      pltpu.VMEM((G, D), jnp.float32)],
  )(agg, agg, cf, cb, h, batch, wlf, wlb, wr, b, wp, bp)


def kernel(x, edge_index, batch, Wlf0, Wrf0, bf0, Wlb0, Wrb0, bb0,
           Wlf1, Wrf1, bf1, Wlb1, Wrb1, bb1, Wp, bp):
  eidx = edge_index.reshape(2 * E)
  zrows = jnp.zeros((BROW, D), jnp.float32)

  agg0, cnt = _sc_aggregate(x, zrows, eidx, with_counts=True)
  cf = cnt[:HR].reshape(HR * D)[:N, None]
  cb = cnt[HR:].reshape(HR * D)[:N, None]
  h1 = _layer(agg0, cf, cb, x,
              Wlf0 * 0.5, Wlb0 * 0.5, (Wrf0 + Wrb0) * 0.5,
              ((bf0 + bb0) * 0.5)[None, :])
  (agg1,) = _sc_aggregate(h1, zrows, eidx, with_counts=False)
  return _final(agg1, cf, cb, h1, batch[:, None],
                Wlf1 * 0.5, Wlb1 * 0.5, (Wrf1 + Wrb1) * 0.5,
                ((bf1 + bb1) * 0.5)[None, :], Wp, bp[None, :])
